# baseline (speedup 1.0000x reference)
import functools

import jax
import jax.numpy as jnp
from jax import lax
from jax.experimental import pallas as pl
from jax.experimental.pallas import tpu as pltpu
from jax.experimental.pallas import tpu_sc as plsc

LANES = 128
SUB = 8


@functools.lru_cache(maxsize=None)
def _make_gather(batch: int, seq: int, emb: int):
    info = plsc.get_sparse_core_info()
    nc, ns = info.num_cores, info.num_subcores
    nw = nc * ns
    assert batch == nw * LANES and emb % SUB == 0
    etiles = emb // SUB

    mesh = plsc.VectorSubcoreMesh(core_axis_name="c", subcore_axis_name="s")

    @functools.partial(
        pl.kernel,
        mesh=mesh,
        out_type=jax.ShapeDtypeStruct((seq, etiles, nw, SUB, LANES), jnp.float32),
        scratch_types=[
            pltpu.VMEM((seq, LANES), jnp.int32),
            pltpu.VMEM((2, LANES, emb), jnp.float32),
            pltpu.VMEM((2, etiles, SUB, LANES), jnp.float32),
            pltpu.SemaphoreType.DMA,
            pltpu.SemaphoreType.DMA,
        ],
        compiler_params=pltpu.CompilerParams(
            use_tc_tiling_on_sc=False, needs_layout_passes=False
        ),
    )
    def gather_kernel(idx_hbm, table_hbm, out_hbm, idx_v, r_v, s_v, gsem, wsem):
        w = lax.axis_index("s") * nc + lax.axis_index("c")
        pltpu.sync_copy(idx_hbm.at[w], idx_v)
        iota = lax.iota(jnp.int32, 16)

        def start_gather(s, buf):
            pltpu.async_copy(table_hbm.at[idx_v.at[s]], r_v.at[buf], gsem)

        def wait_gather(buf):
            pltpu.make_async_copy(table_hbm.at[idx_v.at[0]], r_v.at[buf], gsem).wait()

        def transpose(buf):
            for blg in range(LANES // 16):
                row = iota + 16 * blg
                for e in range(emb):
                    col = jnp.full((16,), e, jnp.int32)
                    v = plsc.load_gather(r_v.at[buf], [row, col])
                    s_v[buf, e // SUB, e % SUB, pl.ds(16 * blg, 16)] = v

        def start_write(s, buf):
            for et in range(etiles):
                pltpu.async_copy(s_v.at[buf, et], out_hbm.at[s, et, w], wsem)

        def wait_write(buf):
            for et in range(etiles):
                pltpu.make_async_copy(
                    out_hbm.at[0, et, 0], s_v.at[buf, et], wsem
                ).wait()

        def step(s, buf, first, last):
            wait_gather(buf)

            @pl.when(jnp.logical_not(last))
            def _():
                start_gather(s + 1, 1 - buf)

            @pl.when(jnp.logical_not(first))
            def _():
                wait_write(buf)

            transpose(buf)
            start_write(s, buf)

        start_gather(0, 0)

        def body(s2, carry):
            s = 2 * s2
            step(s, 0, s2 == 0, jnp.bool_(False))
            step(s + 1, 1, s2 == 0, s2 == seq // 2 - 1)
            return carry

        lax.fori_loop(0, seq // 2, body, 0, unroll=False)
        wait_write(0)
        wait_write(1)

    def run(x2d, table):
        xt = x2d.reshape(nw, LANES, seq).transpose(0, 2, 1)
        return gather_kernel(xt, table)

    return run


def kernel(x, W):
    batch, seq = x.shape
    emb = W.shape[1]
    run = _make_gather(batch, seq, emb)
    out5 = run(x.astype(jnp.int32), W)
    nw = out5.shape[2]
    return out5.transpose(2, 4, 0, 1, 3).reshape(batch, seq, emb)


# speedup vs baseline: 1.5066x; 1.5066x over previous
import functools

import jax
import jax.numpy as jnp
from jax import lax
from jax.experimental import pallas as pl
from jax.experimental.pallas import tpu as pltpu
from jax.experimental.pallas import tpu_sc as plsc

LANES = 128
SUB = 8


@functools.lru_cache(maxsize=None)
def _make_gather(batch: int, seq: int, emb: int):
    info = plsc.get_sparse_core_info()
    nc, ns = info.num_cores, info.num_subcores
    nw = nc * ns
    assert batch == nw * LANES and emb % SUB == 0
    etiles = emb // SUB

    mesh = plsc.VectorSubcoreMesh(core_axis_name="c", subcore_axis_name="s")

    @functools.partial(
        pl.kernel,
        mesh=mesh,
        out_type=jax.ShapeDtypeStruct((seq, etiles, nw, SUB, LANES), jnp.float32),
        scratch_types=[
            pltpu.VMEM((seq, LANES), jnp.int32),
            pltpu.VMEM((2, LANES, emb), jnp.float32),
            pltpu.VMEM((2, etiles, SUB, LANES), jnp.float32),
            pltpu.SemaphoreType.DMA,
            pltpu.SemaphoreType.DMA,
        ],
        compiler_params=pltpu.CompilerParams(
            use_tc_tiling_on_sc=False, needs_layout_passes=False
        ),
    )
    def gather_kernel(idx_hbm, table_hbm, out_hbm, idx_v, r_v, s_v, gsem, wsem):
        w = lax.axis_index("s") * nc + lax.axis_index("c")
        pltpu.sync_copy(idx_hbm.at[w], idx_v)
        iota = lax.iota(jnp.int32, 16)

        def start_gather(s, buf):
            pltpu.async_copy(table_hbm.at[idx_v.at[s]], r_v.at[buf], gsem)

        def wait_gather(buf):
            pltpu.make_async_copy(table_hbm.at[idx_v.at[0]], r_v.at[buf], gsem).wait()

        def transpose(buf):
            for blg in range(LANES // 16):
                row = iota + 16 * blg
                for eg in range(emb // 16):
                    vals = []
                    for i in range(16):
                        col = jnp.full((16,), 16 * eg + i, jnp.int32)
                        vals.append(plsc.load_gather(r_v.at[buf], [row, col]))
                    for i in range(16):
                        e = 16 * eg + i
                        s_v[buf, e // SUB, e % SUB, pl.ds(16 * blg, 16)] = vals[i]

        def start_write(s, buf):
            for et in range(etiles):
                pltpu.async_copy(s_v.at[buf, et], out_hbm.at[s, et, w], wsem)

        def wait_write(buf):
            for et in range(etiles):
                pltpu.make_async_copy(
                    out_hbm.at[0, et, 0], s_v.at[buf, et], wsem
                ).wait()

        def step(s, buf, first, last):
            wait_gather(buf)

            @pl.when(jnp.logical_not(last))
            def _():
                start_gather(s + 1, 1 - buf)

            @pl.when(jnp.logical_not(first))
            def _():
                wait_write(buf)

            transpose(buf)
            start_write(s, buf)

        start_gather(0, 0)

        def body(s2, carry):
            s = 2 * s2
            step(s, 0, s2 == 0, jnp.bool_(False))
            step(s + 1, 1, s2 == 0, s2 == seq // 2 - 1)
            return carry

        lax.fori_loop(0, seq // 2, body, 0, unroll=False)
        wait_write(0)
        wait_write(1)

    def run(x2d, table):
        xt = x2d.reshape(nw, LANES, seq).transpose(0, 2, 1)
        return gather_kernel(xt, table)

    return run


def kernel(x, W):
    batch, seq = x.shape
    emb = W.shape[1]
    run = _make_gather(batch, seq, emb)
    out5 = run(x.astype(jnp.int32), W)
    nw = out5.shape[2]
    return out5.transpose(2, 4, 0, 1, 3).reshape(batch, seq, emb)


# speedup vs baseline: 2.3878x; 1.5849x over previous
import functools

import jax
import jax.numpy as jnp
from jax import lax
from jax.experimental import pallas as pl
from jax.experimental.pallas import tpu as pltpu
from jax.experimental.pallas import tpu_sc as plsc

LANES = 128
SUB = 8


@functools.lru_cache(maxsize=None)
def _make_gather(batch: int, seq: int, emb: int):
    info = plsc.get_sparse_core_info()
    nc, ns = info.num_cores, info.num_subcores
    nw = nc * ns
    assert batch == nw * LANES and emb % 16 == 0 and seq % SUB == 0
    etiles = emb // SUB
    stiles = seq // SUB

    mesh = plsc.VectorSubcoreMesh(core_axis_name="c", subcore_axis_name="s")

    @functools.partial(
        pl.kernel,
        mesh=mesh,
        out_type=jax.ShapeDtypeStruct((seq, etiles, nw, SUB * LANES), jnp.float32),
        scratch_types=[
            pltpu.VMEM((2, SUB, LANES), jnp.int32),
            pltpu.VMEM((2, LANES, emb), jnp.float32),
            pltpu.VMEM((2, etiles * SUB * LANES), jnp.float32),
            pltpu.SemaphoreType.DMA,
            pltpu.SemaphoreType.DMA,
            pltpu.SemaphoreType.DMA,
        ],
        compiler_params=pltpu.CompilerParams(
            use_tc_tiling_on_sc=False, needs_layout_passes=False
        ),
    )
    def gather_kernel(idx_hbm, table_hbm, out_hbm, idx_v, r_v, s_v, isem, gsem, wsem):
        w = lax.axis_index("s") * nc + lax.axis_index("c")
        iota = lax.iota(jnp.int32, 16)

        def start_idx(st, buf):
            pltpu.async_copy(idx_hbm.at[st, w], idx_v.at[buf], isem)

        def wait_idx(buf):
            pltpu.make_async_copy(idx_hbm.at[0, 0], idx_v.at[buf], isem).wait()

        def start_gather(s, buf):
            ib = (s // SUB) % 2
            pltpu.async_copy(
                table_hbm.at[idx_v.at[ib, s % SUB]], r_v.at[buf], gsem
            )

        def wait_gather(buf):
            pltpu.make_async_copy(
                table_hbm.at[idx_v.at[0, 0]], r_v.at[buf], gsem
            ).wait()

        def transpose(buf):
            def tbody(blg, carry):
                row = iota + (blg << 4)
                for eg in range(emb // 16):
                    vals = []
                    for d in range(16):
                        col = ((iota + d) & 15) + 16 * eg
                        vals.append((col, plsc.load_gather(r_v.at[buf], [row, col])))
                    for col, v in vals:
                        sidx = (col << 7) + row
                        plsc.store_scatter(s_v.at[buf], [sidx], v)
                return carry

            lax.fori_loop(0, LANES // 16, tbody, 0, unroll=False)

        def start_write(s, buf):
            for et in range(etiles):
                pltpu.async_copy(
                    s_v.at[buf, pl.ds(et * SUB * LANES, SUB * LANES)],
                    out_hbm.at[s, et, w],
                    wsem,
                )

        def wait_write(buf):
            for et in range(etiles):
                pltpu.make_async_copy(
                    out_hbm.at[0, et, 0],
                    s_v.at[buf, pl.ds(et * SUB * LANES, SUB * LANES)],
                    wsem,
                ).wait()

        def step(s, buf, first, last):
            st = s // SUB
            ss = s % SUB
            wait_gather(buf)

            @pl.when(jnp.logical_and(ss == 0, st < stiles - 1))
            def _():
                start_idx(st + 1, (st + 1) % 2)

            @pl.when(jnp.logical_and(ss == SUB - 2, st < stiles - 1))
            def _():
                wait_idx((st + 1) % 2)

            @pl.when(jnp.logical_not(last))
            def _():
                start_gather(s + 1, 1 - buf)

            @pl.when(jnp.logical_not(first))
            def _():
                wait_write(buf)

            transpose(buf)
            start_write(s, buf)

        pltpu.sync_copy(idx_hbm.at[0, w], idx_v.at[0])
        start_gather(0, 0)

        def body(s2, carry):
            s = 2 * s2
            step(s, 0, s2 == 0, jnp.bool_(False))
            step(s + 1, 1, s2 == 0, s2 == seq // 2 - 1)
            return carry

        lax.fori_loop(0, seq // 2, body, 0, unroll=False)
        wait_write(0)
        wait_write(1)

    def run(x2d, table):
        x4 = (
            x2d.T.reshape(stiles, SUB, nw, LANES).transpose(0, 2, 1, 3)
        )
        return gather_kernel(x4, table)

    return run


def kernel(x, W):
    batch, seq = x.shape
    emb = W.shape[1]
    run = _make_gather(batch, seq, emb)
    out4 = run(x.astype(jnp.int32), W)
    nw = out4.shape[2]
    out5 = out4.reshape(seq, emb // SUB, nw, SUB, LANES)
    return out5.transpose(2, 4, 0, 1, 3).reshape(batch, seq, emb)


# speedup vs baseline: 3.6192x; 1.5157x over previous
import functools

import jax
import jax.numpy as jnp
from jax import lax
from jax.experimental import pallas as pl
from jax.experimental.pallas import tpu as pltpu
from jax.experimental.pallas import tpu_sc as plsc

LANES = 128
SUB = 8


@functools.lru_cache(maxsize=None)
def _make_gather(batch: int, seq: int, emb: int, vocab: int):
    info = plsc.get_sparse_core_info()
    nc, ns = info.num_cores, info.num_subcores
    nw = nc * ns
    assert batch == nw * LANES and emb % 16 == 0 and seq % SUB == 0
    etiles = emb // SUB
    stiles = seq // SUB

    mesh = plsc.VectorSubcoreMesh(core_axis_name="c", subcore_axis_name="s")

    RPK = 384
    RMAIN = (vocab // RPK) * RPK
    RTAIL = vocab - RMAIN

    @functools.partial(
        pl.kernel,
        mesh=mesh,
        out_type=jax.ShapeDtypeStruct((vocab * emb,), jnp.float32),
        scratch_types=[
            pltpu.VMEM((emb, RPK), jnp.float32),
            pltpu.VMEM((emb, RPK), jnp.float32),
            pltpu.VMEM((emb * RPK,), jnp.float32),
            pltpu.VMEM((emb * RPK,), jnp.float32),
            pltpu.SemaphoreType.DMA,
            pltpu.SemaphoreType.DMA,
        ],
        compiler_params=pltpu.CompilerParams(needs_layout_passes=False),
    )
    def repack_kernel(wt_hbm, out_hbm, in_0, in_1, out_0, out_1, isem, osem):
        w = lax.axis_index("s") * nc + lax.axis_index("c")
        iota = lax.iota(jnp.int32, 16)
        nchunks = RMAIN // RPK
        n_t = (nchunks - 1 - w) // nw + 1

        def start_in(t, in_b):
            pltpu.async_copy(
                wt_hbm.at[:, pl.ds((w + nw * t) * RPK, RPK)], in_b, isem
            )

        def wait_in(in_b):
            pltpu.make_async_copy(wt_hbm.at[:, pl.ds(0, RPK)], in_b, isem).wait()

        def wait_out(out_b):
            pltpu.make_async_copy(
                out_hbm.at[pl.ds(0, emb * RPK)], out_b, osem
            ).wait()

        def transpose_chunk(in_b, out_b):
            def tbody(vg, carry):
                vvec = (vg << 4) + iota
                vbase = vvec << 6
                for eg in range(emb // 16):
                    vals = []
                    for d in range(16):
                        evec = ((iota + d) & 15) + 16 * eg
                        vals.append(
                            (evec, plsc.load_gather(in_b, [evec, vvec]))
                        )
                    for evec, v in vals:
                        plsc.store_scatter(out_b, [vbase + evec], v)
                return carry

            lax.fori_loop(0, RPK // 16, tbody, 0, unroll=False)

        def do_chunk(t, in_b, out_b, nxt_in):
            wait_in(in_b)

            @pl.when(t + 1 < n_t)
            def _():
                start_in(t + 1, nxt_in)

            @pl.when(t >= 2)
            def _():
                wait_out(out_b)

            transpose_chunk(in_b, out_b)
            pltpu.async_copy(
                out_b,
                out_hbm.at[pl.ds((w + nw * t) * RPK * emb, RPK * emb)],
                osem,
            )

        start_in(0, in_0)

        def body(t, carry):
            @pl.when(t % 2 == 0)
            def _():
                do_chunk(t, in_0, out_0, in_1)

            @pl.when(t % 2 == 1)
            def _():
                do_chunk(t, in_1, out_1, in_0)

            return carry

        lax.fori_loop(0, n_t, body, 0, unroll=False)
        wait_out(out_0)
        wait_out(out_1)

    @functools.partial(
        pl.kernel,
        mesh=mesh,
        out_type=jax.ShapeDtypeStruct((seq, etiles, nw, SUB * LANES), jnp.float32),
        scratch_types=[
            pltpu.VMEM((2, SUB, LANES), jnp.int32),
            pltpu.VMEM((2, LANES, emb), jnp.float32),
            pltpu.VMEM((2, etiles * SUB * LANES), jnp.float32),
            pltpu.SemaphoreType.DMA,
            pltpu.SemaphoreType.DMA,
            pltpu.SemaphoreType.DMA,
        ],
        compiler_params=pltpu.CompilerParams(
            use_tc_tiling_on_sc=False, needs_layout_passes=False
        ),
    )
    def gather_kernel(idx_hbm, table_hbm, out_hbm, idx_v, r_v, s_v, isem, gsem, wsem):
        w = lax.axis_index("s") * nc + lax.axis_index("c")
        iota = lax.iota(jnp.int32, 16)

        def start_idx(st, buf):
            pltpu.async_copy(idx_hbm.at[st, w], idx_v.at[buf], isem)

        def wait_idx(buf):
            pltpu.make_async_copy(idx_hbm.at[0, 0], idx_v.at[buf], isem).wait()

        def start_gather(s, buf):
            ib = (s // SUB) % 2
            pltpu.async_copy(
                table_hbm.at[idx_v.at[ib, s % SUB]], r_v.at[buf], gsem
            )

        def wait_gather(buf):
            pltpu.make_async_copy(
                table_hbm.at[idx_v.at[0, 0]], r_v.at[buf], gsem
            ).wait()

        def transpose(buf):
            def tbody(blg, carry):
                row = iota + (blg << 4)
                for eg in range(emb // 16):
                    vals = []
                    for d in range(16):
                        col = ((iota + d) & 15) + 16 * eg
                        vals.append((col, plsc.load_gather(r_v.at[buf], [row, col])))
                    for col, v in vals:
                        sidx = (col << 7) + row
                        plsc.store_scatter(s_v.at[buf], [sidx], v)
                return carry

            lax.fori_loop(0, LANES // 16, tbody, 0, unroll=False)

        def start_write(s, buf):
            for et in range(etiles):
                pltpu.async_copy(
                    s_v.at[buf, pl.ds(et * SUB * LANES, SUB * LANES)],
                    out_hbm.at[s, et, w],
                    wsem,
                )

        def wait_write(buf):
            for et in range(etiles):
                pltpu.make_async_copy(
                    out_hbm.at[0, et, 0],
                    s_v.at[buf, pl.ds(et * SUB * LANES, SUB * LANES)],
                    wsem,
                ).wait()

        def step(s, buf, first, last):
            st = s // SUB
            ss = s % SUB
            wait_gather(buf)

            @pl.when(jnp.logical_and(ss == 0, st < stiles - 1))
            def _():
                start_idx(st + 1, (st + 1) % 2)

            @pl.when(jnp.logical_and(ss == SUB - 2, st < stiles - 1))
            def _():
                wait_idx((st + 1) % 2)

            @pl.when(jnp.logical_not(last))
            def _():
                start_gather(s + 1, 1 - buf)

            @pl.when(jnp.logical_not(first))
            def _():
                wait_write(buf)

            transpose(buf)
            start_write(s, buf)

        pltpu.sync_copy(idx_hbm.at[0, w], idx_v.at[0])
        start_gather(0, 0)

        def body(s2, carry):
            s = 2 * s2
            step(s, 0, s2 == 0, jnp.bool_(False))
            step(s + 1, 1, s2 == 0, s2 == seq // 2 - 1)
            return carry

        lax.fori_loop(0, seq // 2, body, 0, unroll=False)
        wait_write(0)
        wait_write(1)

    def run(x2d, table):
        vocab_ = table.shape[0]
        flat = repack_kernel(table.T)
        if RTAIL:
            flat = lax.dynamic_update_slice(
                flat, table[RMAIN:, :].reshape(RTAIL * emb), (RMAIN * emb,)
            )
        table_lin = flat.reshape(vocab_, emb)
        x4 = (
            x2d.T.reshape(stiles, SUB, nw, LANES).transpose(0, 2, 1, 3)
        )
        return gather_kernel(x4, table_lin)

    return run


def kernel(x, W):
    batch, seq = x.shape
    emb = W.shape[1]
    run = _make_gather(batch, seq, emb, W.shape[0])
    out4 = run(x.astype(jnp.int32), W)
    nw = out4.shape[2]
    out5 = out4.reshape(seq, emb // SUB, nw, SUB, LANES)
    return out5.transpose(2, 4, 0, 1, 3).reshape(batch, seq, emb)


# speedup vs baseline: 3.6250x; 1.0016x over previous
import functools

import jax
import jax.numpy as jnp
from jax import lax
from jax.experimental import pallas as pl
from jax.experimental.pallas import tpu as pltpu
from jax.experimental.pallas import tpu_sc as plsc

LANES = 128
SUB = 8


@functools.lru_cache(maxsize=None)
def _make_gather(batch: int, seq: int, emb: int, vocab: int):
    info = plsc.get_sparse_core_info()
    nc, ns = info.num_cores, info.num_subcores
    nw = nc * ns
    assert batch == nw * LANES and emb % 16 == 0 and seq % SUB == 0
    etiles = emb // SUB
    stiles = seq // SUB

    mesh = plsc.VectorSubcoreMesh(core_axis_name="c", subcore_axis_name="s")

    RPK = 384
    RMAIN = (vocab // RPK) * RPK
    RTAIL = vocab - RMAIN

    @functools.partial(
        pl.kernel,
        mesh=mesh,
        out_type=jax.ShapeDtypeStruct((vocab * emb,), jnp.float32),
        scratch_types=[
            pltpu.VMEM((emb, RPK), jnp.float32),
            pltpu.VMEM((emb, RPK), jnp.float32),
            pltpu.VMEM((emb * RPK,), jnp.float32),
            pltpu.VMEM((emb * RPK,), jnp.float32),
            pltpu.SemaphoreType.DMA,
            pltpu.SemaphoreType.DMA,
        ],
        compiler_params=pltpu.CompilerParams(needs_layout_passes=False),
    )
    def repack_kernel(wt_hbm, out_hbm, in_0, in_1, out_0, out_1, isem, osem):
        w = lax.axis_index("s") * nc + lax.axis_index("c")
        iota = lax.iota(jnp.int32, 16)
        nchunks = RMAIN // RPK
        n_t = (nchunks - 1 - w) // nw + 1

        def start_in(t, in_b):
            pltpu.async_copy(
                wt_hbm.at[:, pl.ds((w + nw * t) * RPK, RPK)], in_b, isem
            )

        def wait_in(in_b):
            pltpu.make_async_copy(wt_hbm.at[:, pl.ds(0, RPK)], in_b, isem).wait()

        def wait_out(out_b):
            pltpu.make_async_copy(
                out_hbm.at[pl.ds(0, emb * RPK)], out_b, osem
            ).wait()

        rot = [((iota + d) & 15) for d in range(16)]

        def transpose_chunk(in_b, out_b):
            def tbody(vg, carry):
                vvec = (vg << 4) + iota
                vbase = vvec << 6
                for eg in range(emb // 16):
                    vals = []
                    for d in range(16):
                        evec = rot[d] + 16 * eg
                        vals.append(
                            (evec, plsc.load_gather(in_b, [evec, vvec]))
                        )
                    for evec, v in vals:
                        plsc.store_scatter(out_b, [vbase + evec], v)
                return carry

            lax.fori_loop(0, RPK // 16, tbody, 0, unroll=False)

        def do_chunk(t, in_b, out_b, nxt_in):
            wait_in(in_b)

            @pl.when(t + 1 < n_t)
            def _():
                start_in(t + 1, nxt_in)

            @pl.when(t >= 2)
            def _():
                wait_out(out_b)

            transpose_chunk(in_b, out_b)
            pltpu.async_copy(
                out_b,
                out_hbm.at[pl.ds((w + nw * t) * RPK * emb, RPK * emb)],
                osem,
            )

        start_in(0, in_0)

        def body(t, carry):
            @pl.when(t % 2 == 0)
            def _():
                do_chunk(t, in_0, out_0, in_1)

            @pl.when(t % 2 == 1)
            def _():
                do_chunk(t, in_1, out_1, in_0)

            return carry

        lax.fori_loop(0, n_t, body, 0, unroll=False)
        wait_out(out_0)
        wait_out(out_1)

    @functools.partial(
        pl.kernel,
        mesh=mesh,
        out_type=jax.ShapeDtypeStruct((seq, etiles, nw, SUB * LANES), jnp.float32),
        scratch_types=[
            pltpu.VMEM((2, SUB, LANES), jnp.int32),
            pltpu.VMEM((2, LANES, emb), jnp.float32),
            pltpu.VMEM((2, etiles * SUB * LANES), jnp.float32),
            pltpu.SemaphoreType.DMA,
            pltpu.SemaphoreType.DMA,
            pltpu.SemaphoreType.DMA,
        ],
        compiler_params=pltpu.CompilerParams(
            use_tc_tiling_on_sc=False, needs_layout_passes=False
        ),
    )
    def gather_kernel(idx_hbm, table_hbm, out_hbm, idx_v, r_v, s_v, isem, gsem, wsem):
        w = lax.axis_index("s") * nc + lax.axis_index("c")
        iota = lax.iota(jnp.int32, 16)

        def start_idx(st, buf):
            pltpu.async_copy(idx_hbm.at[st, w], idx_v.at[buf], isem)

        def wait_idx(buf):
            pltpu.make_async_copy(idx_hbm.at[0, 0], idx_v.at[buf], isem).wait()

        def start_gather(s, buf):
            ib = (s // SUB) % 2
            pltpu.async_copy(
                table_hbm.at[idx_v.at[ib, s % SUB]], r_v.at[buf], gsem
            )

        def wait_gather(buf):
            pltpu.make_async_copy(
                table_hbm.at[idx_v.at[0, 0]], r_v.at[buf], gsem
            ).wait()

        def transpose(buf):
            def tbody(blg, carry):
                row = iota + (blg << 4)
                for eg in range(emb // 16):
                    vals = []
                    for d in range(16):
                        col = ((iota + d) & 15) + 16 * eg
                        vals.append((col, plsc.load_gather(r_v.at[buf], [row, col])))
                    for col, v in vals:
                        sidx = (col << 7) + row
                        plsc.store_scatter(s_v.at[buf], [sidx], v)
                return carry

            lax.fori_loop(0, LANES // 16, tbody, 0, unroll=False)

        def start_write(s, buf):
            for et in range(etiles):
                pltpu.async_copy(
                    s_v.at[buf, pl.ds(et * SUB * LANES, SUB * LANES)],
                    out_hbm.at[s, et, w],
                    wsem,
                )

        def wait_write(buf):
            for et in range(etiles):
                pltpu.make_async_copy(
                    out_hbm.at[0, et, 0],
                    s_v.at[buf, pl.ds(et * SUB * LANES, SUB * LANES)],
                    wsem,
                ).wait()

        def step(s, buf, first, last):
            st = s // SUB
            ss = s % SUB
            wait_gather(buf)

            @pl.when(jnp.logical_and(ss == 0, st < stiles - 1))
            def _():
                start_idx(st + 1, (st + 1) % 2)

            @pl.when(jnp.logical_and(ss == SUB - 2, st < stiles - 1))
            def _():
                wait_idx((st + 1) % 2)

            @pl.when(jnp.logical_not(last))
            def _():
                start_gather(s + 1, 1 - buf)

            @pl.when(jnp.logical_not(first))
            def _():
                wait_write(buf)

            transpose(buf)
            start_write(s, buf)

        pltpu.sync_copy(idx_hbm.at[0, w], idx_v.at[0])
        start_gather(0, 0)

        def body(s2, carry):
            s = 2 * s2
            step(s, 0, s2 == 0, jnp.bool_(False))
            step(s + 1, 1, s2 == 0, s2 == seq // 2 - 1)
            return carry

        lax.fori_loop(0, seq // 2, body, 0, unroll=False)
        wait_write(0)
        wait_write(1)

    def run(x2d, table):
        vocab_ = table.shape[0]
        flat = repack_kernel(table.T)
        if RTAIL:
            flat = lax.dynamic_update_slice(
                flat, table[RMAIN:, :].reshape(RTAIL * emb), (RMAIN * emb,)
            )
        table_lin = flat.reshape(vocab_, emb)
        x4 = (
            x2d.T.reshape(stiles, SUB, nw, LANES).transpose(0, 2, 1, 3)
        )
        return gather_kernel(x4, table_lin)

    return run


def kernel(x, W):
    batch, seq = x.shape
    emb = W.shape[1]
    run = _make_gather(batch, seq, emb, W.shape[0])
    out4 = run(x.astype(jnp.int32), W)
    nw = out4.shape[2]
    out5 = out4.reshape(seq, emb // SUB, nw, SUB, LANES)
    return out5.transpose(2, 4, 0, 1, 3).reshape(batch, seq, emb)
